# Initial kernel scaffold; baseline (speedup 1.0000x reference)
import jax
import jax.numpy as jnp
from jax.experimental import pallas as pl


def kernel(text, text_remain_idx, text_remain_padding_mask, text_revert_padding_mask):
    raise NotImplementedError("write your pallas kernel here")


# speedup vs baseline: 2.7317x; 2.7317x over previous
import functools

import jax
import jax.numpy as jnp
from jax import lax
from jax.experimental import pallas as pl
from jax.experimental.pallas import tpu as pltpu
from jax.experimental.pallas import tpu_sc as plsc

_NC = 2
_NS = 16
_NW = _NC * _NS

_B = 16
_L1 = 2049
_LR = 1024
_D = 128
_LO = 1025

_RW = (_B * _LR) // _NW
_CHUNK = 128
_NCHUNK = _RW // _CHUNK


def _gather_sc(table, idx3):
    mesh = plsc.VectorSubcoreMesh(core_axis_name="c", subcore_axis_name="s")

    @functools.partial(
        pl.kernel,
        mesh=mesh,
        out_type=jax.ShapeDtypeStruct((_B * _LO, _D), jnp.float32),
        scratch_types=[
            pltpu.VMEM((_NCHUNK, _CHUNK), jnp.int32),
            pltpu.VMEM((_RW, _D), jnp.float32),
            pltpu.VMEM((1, _D), jnp.float32),
            pltpu.SemaphoreType.DMA,
        ],
        compiler_params=pltpu.CompilerParams(use_tc_tiling_on_sc=False),
    )
    def body(table_hbm, idx_hbm, out_hbm, iv, buf, gt, sem):
        c = lax.axis_index("c")
        s = lax.axis_index("s")
        wid = s * _NC + c

        pltpu.sync_copy(idx_hbm.at[wid], iv)

        off = s * _L1 + 1
        for j in range(_NCHUNK):
            for i in range(_CHUNK // 16):
                sl = pl.ds(i * 16, 16)
                iv[j, sl] = iv[j, sl] + off

        copies = [
            pltpu.async_copy(
                table_hbm.at[iv.at[j]],
                buf.at[pl.ds(j * _CHUNK, _CHUNK)],
                sem,
            )
            for j in range(_NCHUNK)
        ]

        @pl.when(c == 0)
        def _():
            pltpu.sync_copy(table_hbm.at[pl.ds(s * _L1, 1)], gt)
            pltpu.sync_copy(gt, out_hbm.at[pl.ds(s * _LO, 1)])

        for cp in copies:
            cp.wait()

        ob = s * _LO + 1 + c * _RW
        pltpu.sync_copy(buf, out_hbm.at[pl.ds(ob, _RW)])

    return body(table, idx3)


def kernel(text, text_remain_idx, text_remain_padding_mask, text_revert_padding_mask):
    table = text.reshape(_B * _L1, _D)
    idx3 = text_remain_idx.astype(jnp.int32).reshape(_NW, _NCHUNK, _CHUNK)
    out = _gather_sc(table, idx3).reshape(_B, _LO, _D)
    ones = jnp.ones((_B, 1), dtype=jnp.float32)
    remain_mask = jnp.concatenate([ones, text_remain_padding_mask], axis=-1)
    revert_mask = jnp.concatenate([ones, text_revert_padding_mask], axis=-1)
    return (out, remain_mask, revert_mask)
